# initial kernel scaffold (unmeasured)
import jax
import jax.numpy as jnp
from jax import lax
from jax.experimental import pallas as pl
from jax.experimental.pallas import tpu as pltpu

B = 32
H = 16
D = 128
BS = 32
NP = 256
NK = NP * BS
SCALE = D ** -0.5


def kernel(Q, K, V, bt, lens):
    lens2 = lens.reshape(B, 1)

    def body(q_ref, k_ref, v_ref, bt_ref, lens_ref, out_ref,
             cnt_ref, acc_ref, stats_ref, racc_ref, rstats_ref,
             send_sems, recv_sems):
        h = pl.program_id(0)
        my_x = lax.axis_index("x")
        my_y = lax.axis_index("y")
        my_z = lax.axis_index("z")
        peer = (my_x, 1 - my_y, my_z)
        bsem = pltpu.get_barrier_semaphore()

        @pl.when(h == 0)
        def _():
            pid = lax.broadcasted_iota(jnp.int32, (NP, B, NP), 0) + my_y * NP
            btv = bt_ref[...]
            jmask = lax.broadcasted_iota(jnp.int32, (B, NP), 1) < lens_ref[...]
            hit = (btv[None, :, :] == pid) & jmask[None, :, :]
            cnt_ref[...] = jnp.sum(hit.astype(jnp.float32), axis=2)

        q2 = q_ref[...].reshape(B, D).astype(jnp.bfloat16)
        k2 = k_ref[...].reshape(NK, D).astype(jnp.bfloat16)
        v2 = v_ref[...].reshape(NK, D).astype(jnp.bfloat16)

        s = lax.dot_general(k2, q2, (((1,), (1,)), ((), ())),
                            preferred_element_type=jnp.float32) * SCALE
        c = cnt_ref[...]
        w = jnp.broadcast_to(c[:, None, :], (NP, BS, B)).reshape(NK, B)
        m = jnp.max(s, axis=0, keepdims=True)
        p = w * jnp.exp(s - m)
        l = jnp.sum(p, axis=0, keepdims=True)
        acc = lax.dot_general(p.astype(jnp.bfloat16), v2,
                              (((0,), (0,)), ((), ())),
                              preferred_element_type=jnp.float32)
        acc_ref[:, h, :] = acc
        stats_ref[0, h, :] = m[0, :]
        stats_ref[1, h, :] = l[0, :]

        @pl.when(h == H - 1)
        def _():
            pl.semaphore_signal(bsem, inc=1, device_id=peer,
                                device_id_type=pl.DeviceIdType.MESH)
            pl.semaphore_wait(bsem, 1)

            rdma_acc = pltpu.make_async_remote_copy(
                src_ref=acc_ref, dst_ref=racc_ref,
                send_sem=send_sems.at[0], recv_sem=recv_sems.at[0],
                device_id=peer, device_id_type=pl.DeviceIdType.MESH)
            rdma_stats = pltpu.make_async_remote_copy(
                src_ref=stats_ref, dst_ref=rstats_ref,
                send_sem=send_sems.at[1], recv_sem=recv_sems.at[1],
                device_id=peer, device_id_type=pl.DeviceIdType.MESH)
            rdma_acc.start()
            rdma_stats.start()
            rdma_acc.wait()
            rdma_stats.wait()

            m_s = stats_ref[0, :, :]
            l_s = stats_ref[1, :, :]
            m_p = rstats_ref[0, :, :]
            l_p = rstats_ref[1, :, :]
            mm = jnp.maximum(m_s, m_p)
            ea = jnp.exp(m_s - mm)
            eb = jnp.exp(m_p - mm)
            lt = l_s * ea + l_p * eb
            ws = (ea / lt).T
            wp = (eb / lt).T
            out_ref[:, 0, :, :] = (acc_ref[...] * ws[:, :, None]
                                   + racc_ref[...] * wp[:, :, None])

    return pl.pallas_call(
        body,
        grid=(H,),
        in_specs=[
            pl.BlockSpec((B, 1, 1, D), lambda h: (0, 0, h, 0)),
            pl.BlockSpec((NP, BS, 1, D), lambda h: (0, 0, h, 0)),
            pl.BlockSpec((NP, BS, 1, D), lambda h: (0, 0, h, 0)),
            pl.BlockSpec((B, NP), lambda h: (0, 0)),
            pl.BlockSpec((B, 1), lambda h: (0, 0)),
        ],
        out_specs=pl.BlockSpec((B, 1, H, D), lambda h: (0, 0, 0, 0)),
        out_shape=jax.ShapeDtypeStruct((B, 1, H, D), jnp.float32),
        scratch_shapes=[
            pltpu.VMEM((NP, B), jnp.float32),
            pltpu.VMEM((B, H, D), jnp.float32),
            pltpu.VMEM((2, H, B), jnp.float32),
            pltpu.VMEM((B, H, D), jnp.float32),
            pltpu.VMEM((2, H, B), jnp.float32),
            pltpu.SemaphoreType.DMA((2,)),
            pltpu.SemaphoreType.DMA((2,)),
        ],
        compiler_params=pltpu.CompilerParams(
            dimension_semantics=("arbitrary",),
            collective_id=0,
        ),
    )(Q, K, V, bt, lens2)


# baseline (device time: 205492 ns/iter reference)
import jax
import jax.numpy as jnp
from jax import lax
from jax.experimental import pallas as pl
from jax.experimental.pallas import tpu as pltpu

B = 32
H = 16
D = 128
BS = 32
NP = 256
NK = NP * BS
SCALE = D ** -0.5


def kernel(Q, K, V, bt, lens):
    lens2 = lens.reshape(B, 1)
    Qr = Q.reshape(B, H * D)
    Kr = K.reshape(NP, BS, H * D)
    Vr = V.reshape(NP, BS, H * D)

    def body(q_ref, k_ref, v_ref, bt_ref, lens_ref, out_ref,
             cnt_ref, acc_ref, stats_ref, racc_ref, rstats_ref,
             send_sems, recv_sems):
        h = pl.program_id(0)
        my_x = lax.axis_index("x")
        my_y = lax.axis_index("y")
        my_z = lax.axis_index("z")
        peer = (my_x, 1 - my_y, my_z)
        bsem = pltpu.get_barrier_semaphore()

        @pl.when(h == 0)
        def _():
            pid = lax.broadcasted_iota(jnp.int32, (NP, B, NP), 0) + my_y * NP
            btv = bt_ref[...]
            jmask = lax.broadcasted_iota(jnp.int32, (B, NP), 1) < lens_ref[...]
            hit = (btv[None, :, :] == pid) & jmask[None, :, :]
            cnt_ref[...] = jnp.sum(hit.astype(jnp.float32), axis=2)

        q2 = q_ref[...].astype(jnp.bfloat16)
        k2 = k_ref[...].reshape(NK, D).astype(jnp.bfloat16)
        v2 = v_ref[...].reshape(NK, D).astype(jnp.bfloat16)

        s = lax.dot_general(k2, q2, (((1,), (1,)), ((), ())),
                            preferred_element_type=jnp.float32) * SCALE
        c = cnt_ref[...]
        w = jnp.broadcast_to(c[:, None, :], (NP, BS, B)).reshape(NK, B)
        m = jnp.max(s, axis=0, keepdims=True)
        p = w * jnp.exp(s - m)
        l = jnp.sum(p, axis=0, keepdims=True)
        acc = lax.dot_general(p.astype(jnp.bfloat16), v2,
                              (((0,), (0,)), ((), ())),
                              preferred_element_type=jnp.float32)
        acc_ref[:, h, :] = acc
        stats_ref[0, h, :] = m[0, :]
        stats_ref[1, h, :] = l[0, :]

        @pl.when(h == H - 1)
        def _():
            pl.semaphore_signal(bsem, inc=1, device_id=peer,
                                device_id_type=pl.DeviceIdType.MESH)
            pl.semaphore_wait(bsem, 1)

            rdma_acc = pltpu.make_async_remote_copy(
                src_ref=acc_ref, dst_ref=racc_ref,
                send_sem=send_sems.at[0], recv_sem=recv_sems.at[0],
                device_id=peer, device_id_type=pl.DeviceIdType.MESH)
            rdma_stats = pltpu.make_async_remote_copy(
                src_ref=stats_ref, dst_ref=rstats_ref,
                send_sem=send_sems.at[1], recv_sem=recv_sems.at[1],
                device_id=peer, device_id_type=pl.DeviceIdType.MESH)
            rdma_acc.start()
            rdma_stats.start()
            rdma_acc.wait()
            rdma_stats.wait()

            m_s = stats_ref[0, :, :]
            l_s = stats_ref[1, :, :]
            m_p = rstats_ref[0, :, :]
            l_p = rstats_ref[1, :, :]
            mm = jnp.maximum(m_s, m_p)
            ea = jnp.exp(m_s - mm)
            eb = jnp.exp(m_p - mm)
            lt = l_s * ea + l_p * eb
            ws = (ea / lt).T
            wp = (eb / lt).T
            out_ref[...] = (acc_ref[...] * ws[:, :, None]
                            + racc_ref[...] * wp[:, :, None])

    res = pl.pallas_call(
        body,
        grid=(H,),
        in_specs=[
            pl.BlockSpec((B, D), lambda h: (0, h)),
            pl.BlockSpec((NP, BS, D), lambda h: (0, 0, h)),
            pl.BlockSpec((NP, BS, D), lambda h: (0, 0, h)),
            pl.BlockSpec((B, NP), lambda h: (0, 0)),
            pl.BlockSpec((B, 1), lambda h: (0, 0)),
        ],
        out_specs=pl.BlockSpec((B, H, D), lambda h: (0, 0, 0)),
        out_shape=jax.ShapeDtypeStruct((B, H, D), jnp.float32),
        scratch_shapes=[
            pltpu.VMEM((NP, B), jnp.float32),
            pltpu.VMEM((B, H, D), jnp.float32),
            pltpu.VMEM((2, H, B), jnp.float32),
            pltpu.VMEM((B, H, D), jnp.float32),
            pltpu.VMEM((2, H, B), jnp.float32),
            pltpu.SemaphoreType.DMA((2,)),
            pltpu.SemaphoreType.DMA((2,)),
        ],
        compiler_params=pltpu.CompilerParams(
            dimension_semantics=("arbitrary",),
            collective_id=0,
        ),
    )(Qr, Kr, Vr, bt, lens2)
    return res.reshape(B, 1, H, D)


# device time: 190772 ns/iter; 1.0772x vs baseline; 1.0772x over previous
import jax
import jax.numpy as jnp
from jax import lax
from jax.experimental import pallas as pl
from jax.experimental.pallas import tpu as pltpu

B = 32
H = 16
D = 128
BS = 32
NP = 256
NK = NP * BS
SCALE = D ** -0.5


def kernel(Q, K, V, bt, lens):
    lens2 = lens.reshape(B, 1)
    Qr = Q.reshape(B, H * D)
    Kr = K.reshape(NP, BS, H * D)
    Vr = V.reshape(NP, BS, H * D)

    def body(q_ref, k_ref, v_ref, bt_ref, lens_ref, out_ref,
             w_ref, acc_ref, stats_ref, racc_ref, rstats_ref,
             send_sems, recv_sems):
        h = pl.program_id(0)
        my_x = lax.axis_index("x")
        my_y = lax.axis_index("y")
        my_z = lax.axis_index("z")
        peer = (my_x, 1 - my_y, my_z)
        bsem = pltpu.get_barrier_semaphore()

        @pl.when(h == 0)
        def _():
            pid = lax.broadcasted_iota(jnp.int32, (B, NP, NP), 1) + my_y * NP
            btv = bt_ref[...]
            jmask = (lax.broadcasted_iota(jnp.int32, (B, 1, NP), 2)
                     < lens_ref[...][:, :, None])
            hit = (btv[:, None, :] == pid) & jmask
            cnt = jnp.sum(hit.astype(jnp.float32), axis=2)
            expand = (lax.broadcasted_iota(jnp.int32, (NP, NK), 1) // BS
                      == lax.broadcasted_iota(jnp.int32, (NP, NK), 0))
            w_ref[...] = lax.dot_general(
                cnt.astype(jnp.bfloat16), expand.astype(jnp.bfloat16),
                (((1,), (0,)), ((), ())),
                preferred_element_type=jnp.float32)

        q2 = q_ref[...].astype(jnp.bfloat16)
        k2 = k_ref[...].reshape(NK, D).astype(jnp.bfloat16)
        v2 = v_ref[...].reshape(NK, D).astype(jnp.bfloat16)

        s = lax.dot_general(q2, k2, (((1,), (1,)), ((), ())),
                            preferred_element_type=jnp.float32) * SCALE
        m = jnp.max(s, axis=1, keepdims=True)
        p = w_ref[...] * jnp.exp(s - m)
        l = jnp.sum(p, axis=1, keepdims=True)
        acc = lax.dot_general(p.astype(jnp.bfloat16), v2,
                              (((1,), (0,)), ((), ())),
                              preferred_element_type=jnp.float32)
        acc_ref[:, h, :] = acc
        hcol = lax.broadcasted_iota(jnp.int32, (B, H), 1) == h
        stats_ref[0, :, :] = jnp.where(hcol, m, stats_ref[0, :, :])
        stats_ref[1, :, :] = jnp.where(hcol, l, stats_ref[1, :, :])

        @pl.when(h == H - 1)
        def _():
            pl.semaphore_signal(bsem, inc=1, device_id=peer,
                                device_id_type=pl.DeviceIdType.MESH)
            pl.semaphore_wait(bsem, 1)

            rdma_acc = pltpu.make_async_remote_copy(
                src_ref=acc_ref, dst_ref=racc_ref,
                send_sem=send_sems.at[0], recv_sem=recv_sems.at[0],
                device_id=peer, device_id_type=pl.DeviceIdType.MESH)
            rdma_stats = pltpu.make_async_remote_copy(
                src_ref=stats_ref, dst_ref=rstats_ref,
                send_sem=send_sems.at[1], recv_sem=recv_sems.at[1],
                device_id=peer, device_id_type=pl.DeviceIdType.MESH)
            rdma_acc.start()
            rdma_stats.start()
            rdma_acc.wait()
            rdma_stats.wait()

            m_s = stats_ref[0, :, :]
            l_s = stats_ref[1, :, :]
            m_p = rstats_ref[0, :, :]
            l_p = rstats_ref[1, :, :]
            mm = jnp.maximum(m_s, m_p)
            ea = jnp.exp(m_s - mm)
            eb = jnp.exp(m_p - mm)
            lt = l_s * ea + l_p * eb
            out_ref[...] = (acc_ref[...] * (ea / lt)[:, :, None]
                            + racc_ref[...] * (eb / lt)[:, :, None])

    res = pl.pallas_call(
        body,
        grid=(H,),
        in_specs=[
            pl.BlockSpec((B, D), lambda h: (0, h)),
            pl.BlockSpec((NP, BS, D), lambda h: (0, 0, h)),
            pl.BlockSpec((NP, BS, D), lambda h: (0, 0, h)),
            pl.BlockSpec((B, NP), lambda h: (0, 0)),
            pl.BlockSpec((B, 1), lambda h: (0, 0)),
        ],
        out_specs=pl.BlockSpec((B, H, D), lambda h: (0, 0, 0)),
        out_shape=jax.ShapeDtypeStruct((B, H, D), jnp.float32),
        scratch_shapes=[
            pltpu.VMEM((B, NK), jnp.float32),
            pltpu.VMEM((B, H, D), jnp.float32),
            pltpu.VMEM((2, B, H), jnp.float32),
            pltpu.VMEM((B, H, D), jnp.float32),
            pltpu.VMEM((2, B, H), jnp.float32),
            pltpu.SemaphoreType.DMA((2,)),
            pltpu.SemaphoreType.DMA((2,)),
        ],
        compiler_params=pltpu.CompilerParams(
            dimension_semantics=("arbitrary",),
            collective_id=0,
        ),
    )(Qr, Kr, Vr, bt, lens2)
    return res.reshape(B, 1, H, D)
